# baseline (speedup 1.0000x reference)
import functools

import jax
import jax.numpy as jnp
from jax import lax
from jax.experimental import pallas as pl
from jax.experimental.pallas import tpu as pltpu
from jax.experimental.pallas import tpu_sc as plsc

DIM = 128
NUM_CORES = 2
NUM_SUBCORES = 16
NUM_WORKERS = NUM_CORES * NUM_SUBCORES
CHUNK = 200
NBUF = 4


def kernel(x, weight):
    b0, b1 = x.shape
    num_idx = b0 * b1
    idx = x.reshape(num_idx).astype(jnp.int32)
    per_worker = num_idx // NUM_WORKERS
    n_chunks = per_worker // CHUNK
    n_groups = n_chunks // NBUF

    mesh = plsc.VectorSubcoreMesh(core_axis_name="c", subcore_axis_name="s")

    @functools.partial(
        pl.kernel,
        mesh=mesh,
        out_type=jax.ShapeDtypeStruct((num_idx, DIM), jnp.float32),
        scratch_types=[
            pltpu.VMEM((per_worker,), jnp.int32),
            pltpu.VMEM((NBUF, CHUNK, DIM), jnp.float32),
            pltpu.SemaphoreType.DMA((NBUF,)),
        ],
    )
    def gather_kernel(table_hbm, idx_hbm, out_hbm, idx_v, rows_v, sems):
        wid = lax.axis_index("s") * NUM_CORES + lax.axis_index("c")
        base = wid * per_worker

        def gather_desc(i, b):
            return pltpu.make_async_copy(
                table_hbm.at[idx_v.at[pl.ds(i * CHUNK, CHUNK)]],
                rows_v.at[b],
                sems.at[b],
            )

        def store(i, b):
            pltpu.sync_copy(rows_v.at[b], out_hbm.at[pl.ds(base + i * CHUNK, CHUNK)])

        pltpu.sync_copy(idx_hbm.at[pl.ds(base, per_worker)], idx_v)

        for b in range(NBUF):
            gather_desc(b, b).start()

        @pl.loop(0, n_groups - 1)
        def _(g):
            for b in range(NBUF):
                i = g * NBUF + b
                gather_desc(i, b).wait()
                store(i, b)
                gather_desc(i + NBUF, b).start()

        for b in range(NBUF):
            i = (n_groups - 1) * NBUF + b
            gather_desc(i, b).wait()
            store(i, b)

    out = gather_kernel(weight, idx)
    return out.reshape(b0, b1, DIM)


# speedup vs baseline: 1.7638x; 1.7638x over previous
import functools

import jax
import jax.numpy as jnp
from jax import lax
from jax.experimental import pallas as pl
from jax.experimental.pallas import tpu as pltpu
from jax.experimental.pallas import tpu_sc as plsc

DIM = 128
NUM_CORES = 2
NUM_SUBCORES = 16
NUM_WORKERS = NUM_CORES * NUM_SUBCORES
NE = 4
NBUF = 4


def kernel(x, weight):
    b0, b1 = x.shape
    num_idx = b0 * b1
    idx = x.reshape(num_idx).astype(jnp.int32)
    chunk = NE * b1
    per_worker = num_idx // NUM_WORKERS
    elems_per_worker = b0 // NUM_WORKERS
    n_chunks = elems_per_worker // NE
    n_groups = n_chunks // NBUF

    mesh = plsc.VectorSubcoreMesh(core_axis_name="c", subcore_axis_name="s")

    @functools.partial(
        pl.kernel,
        mesh=mesh,
        out_type=jax.ShapeDtypeStruct((b0, b1, DIM), jnp.float32),
        scratch_types=[
            pltpu.VMEM((per_worker,), jnp.int32),
            pltpu.VMEM((NBUF, chunk, DIM), jnp.float32),
            pltpu.SemaphoreType.DMA((NBUF,)),
        ],
    )
    def gather_kernel(table_hbm, idx_hbm, out_hbm, idx_v, rows_v, sems):
        wid = lax.axis_index("s") * NUM_CORES + lax.axis_index("c")
        base = wid * per_worker
        ebase = wid * elems_per_worker

        def gather_desc(i, b):
            return pltpu.make_async_copy(
                table_hbm.at[idx_v.at[pl.ds(i * chunk, chunk)]],
                rows_v.at[b],
                sems.at[b],
            )

        def store(i, b):
            for j in range(NE):
                pltpu.sync_copy(
                    rows_v.at[b, pl.ds(j * b1, b1)],
                    out_hbm.at[ebase + i * NE + j],
                )

        pltpu.sync_copy(idx_hbm.at[pl.ds(base, per_worker)], idx_v)

        for b in range(NBUF):
            gather_desc(b, b).start()

        @pl.loop(0, n_groups - 1)
        def _(g):
            for b in range(NBUF):
                i = g * NBUF + b
                gather_desc(i, b).wait()
                store(i, b)
                gather_desc(i + NBUF, b).start()

        for b in range(NBUF):
            i = (n_groups - 1) * NBUF + b
            gather_desc(i, b).wait()
            store(i, b)

    return gather_kernel(weight, idx)
